# Initial kernel scaffold; baseline (speedup 1.0000x reference)
import jax
import jax.numpy as jnp
from jax.experimental import pallas as pl


def kernel(x, batch, gamma, beta, W, b):
    raise NotImplementedError("write your pallas kernel here")


# speedup vs baseline: 5.2445x; 5.2445x over previous
import functools

import jax
import jax.numpy as jnp
from jax import lax
from jax.experimental import pallas as pl
from jax.experimental.pallas import tpu as pltpu
from jax.experimental.pallas import tpu_sc as plsc

N_NODES = 100000
EMB = 128
GRAPHS = 1024
NC, NS = 2, 16
NW = NC * NS
CHUNK = 128
FULL_CHUNKS = N_NODES // CHUNK
TAIL = N_NODES - FULL_CHUNKS * CHUNK
SLOTS = -(-(FULL_CHUNKS + 1) // NW)
ROWS_PER_TILE = GRAPHS // NS
CNT_W = 16


def _sc_segment_sums(x, batch_i32, zsum, zcnt, ones):
    mesh = plsc.VectorSubcoreMesh(
        core_axis_name="c", subcore_axis_name="s",
        num_cores=NC, num_subcores=NS)

    @functools.partial(
        pl.kernel,
        out_type=(
            jax.ShapeDtypeStruct((NC, GRAPHS, EMB), jnp.float32),
            jax.ShapeDtypeStruct((NC, GRAPHS, CNT_W), jnp.float32),
        ),
        mesh=mesh,
        scratch_types=[
            pltpu.VMEM((CHUNK,), jnp.int32),
            pltpu.VMEM((CHUNK, EMB), jnp.float32),
            pltpu.VMEM((CHUNK, CNT_W), jnp.float32),
            pltpu.VMEM((TAIL,), jnp.int32),
            pltpu.VMEM((TAIL, EMB), jnp.float32),
            pltpu.VMEM_SHARED((GRAPHS, EMB), jnp.float32),
            pltpu.VMEM_SHARED((GRAPHS, CNT_W), jnp.float32),
        ],
    )
    def k(x_hbm, b_hbm, zsum_hbm, zcnt_hbm, ones_hbm,
          out_hbm, cntout_hbm,
          idx_v, xb, ones_v, idx_t, xb_t, acc, cnt):
        cid = lax.axis_index("c")
        sid = lax.axis_index("s")
        w = cid * NS + sid
        base = sid * ROWS_PER_TILE
        pltpu.sync_copy(zsum_hbm.at[pl.ds(base, ROWS_PER_TILE)],
                        acc.at[pl.ds(base, ROWS_PER_TILE)])
        pltpu.sync_copy(zcnt_hbm.at[pl.ds(base, ROWS_PER_TILE)],
                        cnt.at[pl.ds(base, ROWS_PER_TILE)])
        pltpu.sync_copy(ones_hbm, ones_v)
        plsc.subcore_barrier()

        def body(j, carry):
            c = w + NW * j

            @pl.when(c < FULL_CHUNKS)
            def _():
                off = c * CHUNK
                pltpu.sync_copy(b_hbm.at[pl.ds(off, CHUNK)], idx_v)
                pltpu.sync_copy(x_hbm.at[pl.ds(off, CHUNK)], xb)
                pltpu.sync_copy(xb, acc.at[idx_v], add=True)
                pltpu.sync_copy(ones_v, cnt.at[idx_v], add=True)
            return carry

        lax.fori_loop(0, SLOTS, body, 0)

        @pl.when(w == 0)
        def _tail():
            off = FULL_CHUNKS * CHUNK
            pltpu.sync_copy(b_hbm.at[pl.ds(off, TAIL)], idx_t)
            pltpu.sync_copy(x_hbm.at[pl.ds(off, TAIL)], xb_t)
            pltpu.sync_copy(xb_t, acc.at[idx_t], add=True)
            pltpu.sync_copy(ones_v.at[pl.ds(0, TAIL)], cnt.at[idx_t], add=True)

        plsc.subcore_barrier()
        pltpu.sync_copy(acc.at[pl.ds(base, ROWS_PER_TILE)],
                        out_hbm.at[cid, pl.ds(base, ROWS_PER_TILE)])
        pltpu.sync_copy(cnt.at[pl.ds(base, ROWS_PER_TILE)],
                        cntout_hbm.at[cid, pl.ds(base, ROWS_PER_TILE)])

    return k(x, batch_i32, zsum, zcnt, ones)


def _tc_head(pool, cnts, gamma, beta, W, b):
    num_tasks = W.shape[1]

    def body(p_ref, c_ref, g_ref, be_ref, w_ref, b_ref, o_ref):
        s = p_ref[0] + p_ref[1]
        cc = c_ref[0] + c_ref[1]
        cnt = cc[:, 0:1]
        rep = s / jnp.maximum(cnt, 1.0)
        mu = jnp.mean(rep, axis=1, keepdims=True)
        var = jnp.mean((rep - mu) ** 2, axis=1, keepdims=True)
        nrm = (rep - mu) * lax.rsqrt(var + 1e-5) * g_ref[...] + be_ref[...]
        o_ref[...] = (jnp.dot(nrm, w_ref[...],
                              preferred_element_type=jnp.float32)
                      + b_ref[...])

    return pl.pallas_call(
        body,
        out_shape=jax.ShapeDtypeStruct((GRAPHS, num_tasks), jnp.float32),
    )(pool, cnts, gamma.reshape(1, EMB), beta.reshape(1, EMB), W,
      b.reshape(1, num_tasks))


def kernel(x, batch, gamma, beta, W, b):
    batch_i32 = batch.astype(jnp.int32)
    zsum = jnp.zeros((GRAPHS, EMB), jnp.float32)
    zcnt = jnp.zeros((GRAPHS, CNT_W), jnp.float32)
    ones = jnp.ones((CHUNK, CNT_W), jnp.float32)
    pool, cnts = _sc_segment_sums(x, batch_i32, zsum, zcnt, ones)
    return _tc_head(pool, cnts, gamma, beta, W, b)


# speedup vs baseline: 8.2164x; 1.5667x over previous
import functools

import jax
import jax.numpy as jnp
from jax import lax
from jax.experimental import pallas as pl
from jax.experimental.pallas import tpu as pltpu
from jax.experimental.pallas import tpu_sc as plsc

N_NODES = 100000
EMB = 128
GRAPHS = 1024
NC, NS = 2, 16
NW = NC * NS
CHUNK = 128
FULL_CHUNKS = N_NODES // CHUNK
TAIL = N_NODES - FULL_CHUNKS * CHUNK
SLOTS = -(-(FULL_CHUNKS + 1) // NW)
ROWS_PER_TILE = GRAPHS // NS
CNT_W = 16
NBUF = 4


def _sc_segment_sums(x, batch_i32, zsum, zcnt, ones):
    mesh = plsc.VectorSubcoreMesh(
        core_axis_name="c", subcore_axis_name="s",
        num_cores=NC, num_subcores=NS)

    @functools.partial(
        pl.kernel,
        out_type=(
            jax.ShapeDtypeStruct((NC, GRAPHS, EMB), jnp.float32),
            jax.ShapeDtypeStruct((NC, GRAPHS, CNT_W), jnp.float32),
        ),
        mesh=mesh,
        scratch_types=[
            pltpu.VMEM((NBUF, CHUNK), jnp.int32),
            pltpu.VMEM((CHUNK, EMB), jnp.float32),
            pltpu.VMEM((CHUNK, EMB), jnp.float32),
            pltpu.VMEM((CHUNK, EMB), jnp.float32),
            pltpu.VMEM((CHUNK, EMB), jnp.float32),
            pltpu.VMEM((CHUNK, CNT_W), jnp.float32),
            pltpu.VMEM((TAIL,), jnp.int32),
            pltpu.VMEM((TAIL, EMB), jnp.float32),
            pltpu.SemaphoreType.DMA,
            pltpu.SemaphoreType.DMA,
            pltpu.SemaphoreType.DMA,
            pltpu.SemaphoreType.DMA,
            pltpu.SemaphoreType.DMA,
            pltpu.SemaphoreType.DMA,
            pltpu.SemaphoreType.DMA,
            pltpu.SemaphoreType.DMA,
            pltpu.VMEM_SHARED((GRAPHS, EMB), jnp.float32),
            pltpu.VMEM_SHARED((GRAPHS, CNT_W), jnp.float32),
        ],
    )
    def k(x_hbm, b_hbm, zsum_hbm, zcnt_hbm, ones_hbm,
          out_hbm, cntout_hbm,
          idxr, xb0, xb1, xb2, xb3, ones_v, idx_t, xb_t,
          lsem0, lsem1, lsem2, lsem3, ssem0, ssem1, ssem2, ssem3, acc, cnt):
        cid = lax.axis_index("c")
        sid = lax.axis_index("s")
        w = cid * NS + sid
        base = sid * ROWS_PER_TILE
        xbs = (xb0, xb1, xb2, xb3)
        lsems = (lsem0, lsem1, lsem2, lsem3)
        ssems = (ssem0, ssem1, ssem2, ssem3)

        def issue_load(j, bnum):
            @pl.when((j >= 0) & (w + NW * j < FULL_CHUNKS))
            def _():
                off = (w + NW * j) * CHUNK
                pltpu.async_copy(b_hbm.at[pl.ds(off, CHUNK)],
                                 idxr.at[bnum], lsems[bnum])
                pltpu.async_copy(x_hbm.at[pl.ds(off, CHUNK)],
                                 xbs[bnum], lsems[bnum])

        def wait_load(j, bnum):
            @pl.when((j >= 0) & (w + NW * j < FULL_CHUNKS))
            def _():
                off = (w + NW * j) * CHUNK
                pltpu.make_async_copy(b_hbm.at[pl.ds(off, CHUNK)],
                                      idxr.at[bnum], lsems[bnum]).wait()
                pltpu.make_async_copy(x_hbm.at[pl.ds(off, CHUNK)],
                                      xbs[bnum], lsems[bnum]).wait()

        def issue_scat(j, bnum):
            @pl.when((j >= 0) & (w + NW * j < FULL_CHUNKS))
            def _():
                pltpu.async_copy(xbs[bnum], acc.at[idxr.at[bnum]],
                                 ssems[bnum], add=True)
                pltpu.async_copy(ones_v, cnt.at[idxr.at[bnum]],
                                 ssems[bnum], add=True)

        def wait_scat(j, bnum):
            @pl.when((j >= 0) & (w + NW * j < FULL_CHUNKS))
            def _():
                pltpu.make_async_copy(xbs[bnum], acc.at[idxr.at[bnum]],
                                      ssems[bnum]).wait()
                pltpu.make_async_copy(ones_v, cnt.at[idxr.at[bnum]],
                                      ssems[bnum]).wait()

        issue_load(0, 0)
        issue_load(1, 1)
        pltpu.sync_copy(zsum_hbm.at[pl.ds(base, ROWS_PER_TILE)],
                        acc.at[pl.ds(base, ROWS_PER_TILE)])
        pltpu.sync_copy(zcnt_hbm.at[pl.ds(base, ROWS_PER_TILE)],
                        cnt.at[pl.ds(base, ROWS_PER_TILE)])
        pltpu.sync_copy(ones_hbm, ones_v)
        plsc.subcore_barrier()

        def body(g, carry):
            for bnum in range(NBUF):
                j = NBUF * g + bnum
                wait_scat(j - 2, (bnum + 2) % NBUF)
                wait_load(j, bnum)
                issue_scat(j, bnum)
                issue_load(j + 2, (bnum + 2) % NBUF)
            return carry

        lax.fori_loop(0, (SLOTS + NBUF + 1) // NBUF, body, 0)

        @pl.when(w == 0)
        def _tail():
            off = FULL_CHUNKS * CHUNK
            pltpu.sync_copy(b_hbm.at[pl.ds(off, TAIL)], idx_t)
            pltpu.sync_copy(x_hbm.at[pl.ds(off, TAIL)], xb_t)
            pltpu.sync_copy(xb_t, acc.at[idx_t], add=True)
            pltpu.sync_copy(ones_v.at[pl.ds(0, TAIL)], cnt.at[idx_t], add=True)

        plsc.subcore_barrier()
        pltpu.sync_copy(acc.at[pl.ds(base, ROWS_PER_TILE)],
                        out_hbm.at[cid, pl.ds(base, ROWS_PER_TILE)])
        pltpu.sync_copy(cnt.at[pl.ds(base, ROWS_PER_TILE)],
                        cntout_hbm.at[cid, pl.ds(base, ROWS_PER_TILE)])

    return k(x, batch_i32, zsum, zcnt, ones)


def _tc_head(pool, cnts, gamma, beta, W, b):
    num_tasks = W.shape[1]

    def body(p_ref, c_ref, g_ref, be_ref, w_ref, b_ref, o_ref):
        s = p_ref[0] + p_ref[1]
        cc = c_ref[0] + c_ref[1]
        cnt = cc[:, 0:1]
        rep = s / jnp.maximum(cnt, 1.0)
        mu = jnp.mean(rep, axis=1, keepdims=True)
        var = jnp.mean((rep - mu) ** 2, axis=1, keepdims=True)
        nrm = (rep - mu) * lax.rsqrt(var + 1e-5) * g_ref[...] + be_ref[...]
        o_ref[...] = (jnp.dot(nrm, w_ref[...],
                              preferred_element_type=jnp.float32)
                      + b_ref[...])

    return pl.pallas_call(
        body,
        out_shape=jax.ShapeDtypeStruct((GRAPHS, num_tasks), jnp.float32),
    )(pool, cnts, gamma.reshape(1, EMB), beta.reshape(1, EMB), W,
      b.reshape(1, num_tasks))


def kernel(x, batch, gamma, beta, W, b):
    batch_i32 = batch.astype(jnp.int32)
    zsum = jnp.zeros((GRAPHS, EMB), jnp.float32)
    zcnt = jnp.zeros((GRAPHS, CNT_W), jnp.float32)
    ones = jnp.ones((CHUNK, CNT_W), jnp.float32)
    pool, cnts = _sc_segment_sums(x, batch_i32, zsum, zcnt, ones)
    return _tc_head(pool, cnts, gamma, beta, W, b)
